# baseline (speedup 1.0000x reference)
import functools

import jax
import jax.numpy as jnp
from jax import lax
from jax.experimental import pallas as pl
from jax.experimental.pallas import tpu as pltpu
from jax.experimental.pallas import tpu_sc as plsc

N = 10000
E = 320000
D = 128
H = 128
DE = 16

NC = 2
NS = 16
NW = NC * NS
EPT = E // NW
K = 80
NCHUNK = EPT // K
DPC = 128
NDC = (EPT + DPC - 1) // DPC
NPAD = 10240


def _proj_body(x_ref, wa_ref, wb_ref, a_ref, b_ref):
    xb = x_ref[...]
    a_ref[...] = jnp.dot(xb, wa_ref[...], preferred_element_type=jnp.float32)
    b_ref[...] = jnp.dot(xb, wb_ref[...], preferred_element_type=jnp.float32)


def _eatt_body(ea_ref, wc_ref, b1_ref, o_ref):
    o_ref[...] = (
        jnp.dot(ea_ref[...], wc_ref[...], preferred_element_type=jnp.float32)
        + b1_ref[...]
    )


def _bf16r(v):
    return v.astype(jnp.bfloat16).astype(jnp.float32)


def _final_body(acc_ref, sw_ref, w2_ref, b2_ref, wr1_ref, br1_ref, wr2_ref,
                br2_ref, o_ref):
    hi = lax.Precision.HIGHEST
    acc = jnp.sum(acc_ref[...], axis=0, keepdims=True)
    sw = jnp.sum(sw_ref[...])
    hg = (jnp.dot(acc, w2_ref[...], preferred_element_type=jnp.float32,
                  precision=hi) + sw * b2_ref[...]) * (1.0 / N)
    h = jnp.maximum(
        jnp.dot(_bf16r(hg), wr1_ref[...], preferred_element_type=jnp.float32,
                precision=hi) + br1_ref[...], 0.0)
    o_ref[...] = (jnp.dot(_bf16r(h), wr2_ref[...],
                          preferred_element_type=jnp.float32, precision=hi)
                  + br2_ref[...])


_MESH = plsc.VectorSubcoreMesh(core_axis_name="c", subcore_axis_name="s",
                               num_cores=NC, num_subcores=NS)


EPTP = NDC * DPC
SLICE = NPAD // NS


@functools.partial(
    pl.kernel,
    out_type=jax.ShapeDtypeStruct((NC, NPAD), jnp.float32),
    mesh=_MESH,
    compiler_params=pltpu.CompilerParams(needs_layout_passes=False),
    scratch_types=[
        pltpu.VMEM((EPTP,), jnp.int32),
        pltpu.VMEM((NPAD,), jnp.float32),
        pltpu.VMEM_SHARED((NS, NPAD), jnp.float32),
        pltpu.VMEM((NS, SLICE), jnp.float32),
        pltpu.VMEM((SLICE,), jnp.float32),
    ],
)
def _deg_kernel(dstp_hbm, deg_hbm, idx_v, hist_v, shared_h, slice_v, out_v):
    cid = lax.axis_index("c")
    sid = lax.axis_index("s")
    wid = cid * NS + sid

    pltpu.sync_copy(dstp_hbm.at[wid], idx_v)

    def zero(i, c):
        hist_v[pl.ds(i * 16, 16)] = jnp.zeros((16,), jnp.float32)
        return c

    lax.fori_loop(0, NPAD // 16, zero, 0)

    ones = jnp.ones((16,), jnp.float32)

    def scat(g, c):
        vals = idx_v[pl.ds(g * 16, 16)]
        plsc.addupdate_scatter(hist_v, [vals], ones)
        return c

    lax.fori_loop(0, EPTP // 16, scat, 0)

    pltpu.sync_copy(hist_v, shared_h.at[sid])
    plsc.subcore_barrier()

    off = sid * SLICE
    for p in range(NS):
        pltpu.sync_copy(shared_h.at[p, pl.ds(off, SLICE)], slice_v.at[p])

    def merge(i, c):
        sl = pl.ds(i * 16, 16)
        acc = slice_v[0, sl]
        for p in range(1, NS):
            acc = acc + slice_v[p, sl]
        out_v[sl] = acc
        return c

    lax.fori_loop(0, SLICE // 16, merge, 0)
    pltpu.sync_copy(out_v, deg_hbm.at[cid, pl.ds(off, SLICE)])


_GDN = lax.GatherDimensionNumbers(
    offset_dims=(), collapsed_slice_dims=(0,), start_index_map=(0,))


def _lane_bcast(vec, j):
    idx = jnp.full((16, 1), j, jnp.int32)
    return lax.gather(vec, idx, _GDN, (1,),
                      mode=lax.GatherScatterMode.PROMISE_IN_BOUNDS)


@functools.partial(
    pl.kernel,
    out_type=(jax.ShapeDtypeStruct((NW, 8, 16), jnp.float32),
              jax.ShapeDtypeStruct((NW, 16), jnp.float32)),
    mesh=_MESH,
    compiler_params=pltpu.CompilerParams(needs_layout_passes=False),
    scratch_types=[
        pltpu.VMEM((EPT,), jnp.int32),
        pltpu.VMEM((EPT,), jnp.int32),
        pltpu.VMEM((NC, NPAD), jnp.float32),
        pltpu.VMEM((NPAD,), jnp.float32),
        pltpu.VMEM((K, D), jnp.float32),
        pltpu.VMEM((K, D), jnp.float32),
        pltpu.VMEM((K, D), jnp.float32),
        pltpu.VMEM((8, 16), jnp.float32),
        pltpu.VMEM((16,), jnp.float32),
        pltpu.SemaphoreType.DMA,
        pltpu.SemaphoreType.DMA,
        pltpu.SemaphoreType.DMA,
    ],
)
def _edge_kernel(src_hbm, dst_hbm, deg_hbm, a_hbm, b_hbm, e_hbm,
                 acc_hbm, sumw_hbm,
                 src_v, dst_v, deg_v, w_v, a_buf, b_buf, e_buf,
                 acc_v, sumw_v, sem_a, sem_b, sem_e):
    cid = lax.axis_index("c")
    sid = lax.axis_index("s")
    wid = cid * NS + sid
    base = wid * EPT

    pltpu.sync_copy(src_hbm.at[pl.ds(base, EPT)], src_v)
    pltpu.sync_copy(dst_hbm.at[pl.ds(base, EPT)], dst_v)
    pltpu.sync_copy(deg_hbm, deg_v)

    def wbody(i, carry):
        d = deg_v[0, pl.ds(i * 16, 16)] + deg_v[1, pl.ds(i * 16, 16)]
        w_v[pl.ds(i * 16, 16)] = jnp.where(
            d > 0.0, 1.0 / jnp.maximum(d, 1.0), 0.0)
        return carry

    lax.fori_loop(0, N // 16, wbody, 0)

    zero16 = jnp.zeros((16,), jnp.float32)
    iota16 = lax.iota(jnp.int32, 16)

    def chunk(c, carry):
        cp_a = pltpu.async_copy(a_hbm.at[src_v.at[pl.ds(c * K, K)]], a_buf,
                                sem_a)
        cp_b = pltpu.async_copy(b_hbm.at[dst_v.at[pl.ds(c * K, K)]], b_buf,
                                sem_b)
        cp_e = pltpu.async_copy(e_hbm.at[pl.ds(base + c * K, K)], e_buf,
                                sem_e)
        cp_a.wait()
        cp_b.wait()
        cp_e.wait()

        def group(g, gcarry):
            accs, comps, sumw = gcarry
            row0 = g * 16
            dvals = plsc.load_gather(dst_v, [iota16 + (c * K + row0)])
            w16 = plsc.load_gather(w_v, [dvals])
            part = [zero16] * 8
            for j in range(16):
                wj = _lane_bcast(w16, j)
                row = row0 + j
                for r in range(8):
                    sl = pl.ds(r * 16, 16)
                    h = jnp.maximum(
                        a_buf[row, sl] + b_buf[row, sl] + e_buf[row, sl], 0.0)
                    u = plsc.bitcast(h, jnp.int32)
                    h = plsc.bitcast((u + 32768) & jnp.int32(-65536),
                                     jnp.float32)
                    part[r] = part[r] + h * wj
            new_a, new_c = [], []
            for r in range(8):
                y = part[r] - comps[r]
                t = accs[r] + y
                new_c.append((t - accs[r]) - y)
                new_a.append(t)
            return tuple(new_a), tuple(new_c), sumw + w16

        return lax.fori_loop(0, K // 16, group, carry)

    accs, comps, sumw = lax.fori_loop(
        0, NCHUNK, chunk,
        (tuple(zero16 for _ in range(8)), tuple(zero16 for _ in range(8)),
         zero16))

    for r in range(8):
        acc_v[r] = accs[r] - comps[r]
    sumw_v[...] = sumw
    pltpu.sync_copy(acc_v, acc_hbm.at[wid])
    pltpu.sync_copy(sumw_v, sumw_hbm.at[wid])


def kernel(x, edge_attr, edge_index, W1, b1, W2, b2, Wr1, br1, Wr2, br2):
    f32 = jnp.float32
    src = edge_index[0]
    dst = edge_index[1]
    W1a = W1[:D]
    W1b = W1[D:2 * D]
    W1c = W1[2 * D:]

    a_mat, b_mat = pl.pallas_call(
        _proj_body,
        grid=(10,),
        in_specs=[
            pl.BlockSpec((N // 10, D), lambda i: (i, 0)),
            pl.BlockSpec((D, D), lambda i: (0, 0)),
            pl.BlockSpec((D, D), lambda i: (0, 0)),
        ],
        out_specs=[
            pl.BlockSpec((N // 10, D), lambda i: (i, 0)),
            pl.BlockSpec((N // 10, D), lambda i: (i, 0)),
        ],
        out_shape=[jax.ShapeDtypeStruct((N, D), f32)] * 2,
    )(x, W1a, W1b)

    eatt = pl.pallas_call(
        _eatt_body,
        grid=(100,),
        in_specs=[
            pl.BlockSpec((E // 100, DE), lambda i: (i, 0)),
            pl.BlockSpec((DE, H), lambda i: (0, 0)),
            pl.BlockSpec((1, H), lambda i: (0, 0)),
        ],
        out_specs=pl.BlockSpec((E // 100, H), lambda i: (i, 0)),
        out_shape=jax.ShapeDtypeStruct((E, H), f32),
    )(edge_attr, W1c, b1.reshape(1, H))

    dst_pad = jnp.concatenate(
        [dst.reshape(NW, EPT),
         jnp.full((NW, EPTP - EPT), N, jnp.int32)], axis=1)
    degp = _deg_kernel(dst_pad)

    acc, sumw = _edge_kernel(src, dst, degp, a_mat, b_mat, eatt)

    out = pl.pallas_call(
        _final_body,
        in_specs=[
            pl.BlockSpec((NW, H), lambda: (0, 0)),
            pl.BlockSpec((NW, 16), lambda: (0, 0)),
            pl.BlockSpec((H, H), lambda: (0, 0)),
            pl.BlockSpec((1, H), lambda: (0, 0)),
            pl.BlockSpec((H, H), lambda: (0, 0)),
            pl.BlockSpec((1, H), lambda: (0, 0)),
            pl.BlockSpec((H, 1), lambda: (0, 0)),
            pl.BlockSpec((1, 1), lambda: (0, 0)),
        ],
        out_specs=pl.BlockSpec((1, 1), lambda: (0, 0)),
        out_shape=jax.ShapeDtypeStruct((1, 1), f32),
    )(acc.reshape(NW, H), sumw, W2, b2.reshape(1, H), Wr1,
      br1.reshape(1, H), Wr2, br2.reshape(1, 1))

    return out[0, 0]


# speedup vs baseline: 1.3507x; 1.3507x over previous
import functools

import jax
import jax.numpy as jnp
from jax import lax
from jax.experimental import pallas as pl
from jax.experimental.pallas import tpu as pltpu
from jax.experimental.pallas import tpu_sc as plsc

N = 10000
E = 320000
D = 128
H = 128
DE = 16

NC = 2
NS = 16
NW = NC * NS
EPT = E // NW
K = 80
NCHUNK = EPT // K
DPC = 128
NDC = (EPT + DPC - 1) // DPC
NPAD = 10240


def _proj_body(x_ref, wa_ref, wb_ref, a_ref, b_ref):
    xb = x_ref[...]
    a_ref[...] = jnp.dot(xb, wa_ref[...], preferred_element_type=jnp.float32)
    b_ref[...] = jnp.dot(xb, wb_ref[...], preferred_element_type=jnp.float32)


def _eatt_body(ea_ref, wc_ref, b1_ref, o_ref):
    o_ref[...] = (
        jnp.dot(ea_ref[...], wc_ref[...], preferred_element_type=jnp.float32)
        + b1_ref[...]
    )


def _bf16r(v):
    return v.astype(jnp.bfloat16).astype(jnp.float32)


def _final_body(acc_ref, sw_ref, w2_ref, b2_ref, wr1_ref, br1_ref, wr2_ref,
                br2_ref, o_ref):
    hi = lax.Precision.HIGHEST
    acc = jnp.sum(acc_ref[...], axis=0, keepdims=True)
    sw = jnp.sum(sw_ref[...])
    hg = (jnp.dot(acc, w2_ref[...], preferred_element_type=jnp.float32,
                  precision=hi) + sw * b2_ref[...]) * (1.0 / N)
    h = jnp.maximum(
        jnp.dot(_bf16r(hg), wr1_ref[...], preferred_element_type=jnp.float32,
                precision=hi) + br1_ref[...], 0.0)
    o_ref[...] = (jnp.dot(_bf16r(h), wr2_ref[...],
                          preferred_element_type=jnp.float32, precision=hi)
                  + br2_ref[...])


_MESH = plsc.VectorSubcoreMesh(core_axis_name="c", subcore_axis_name="s",
                               num_cores=NC, num_subcores=NS)


EPTP = NDC * DPC
SLICE = NPAD // NS


@functools.partial(
    pl.kernel,
    out_type=jax.ShapeDtypeStruct((NC, NPAD), jnp.float32),
    mesh=_MESH,
    compiler_params=pltpu.CompilerParams(needs_layout_passes=False),
    scratch_types=[
        pltpu.VMEM((EPTP,), jnp.int32),
        pltpu.VMEM((NPAD,), jnp.float32),
        pltpu.VMEM_SHARED((NS, NPAD), jnp.float32),
        pltpu.VMEM((NS, SLICE), jnp.float32),
        pltpu.VMEM((SLICE,), jnp.float32),
    ],
)
def _deg_kernel(dstp_hbm, deg_hbm, idx_v, hist_v, shared_h, slice_v, out_v):
    cid = lax.axis_index("c")
    sid = lax.axis_index("s")
    wid = cid * NS + sid

    pltpu.sync_copy(dstp_hbm.at[wid], idx_v)

    def zero(i, c):
        hist_v[pl.ds(i * 16, 16)] = jnp.zeros((16,), jnp.float32)
        return c

    lax.fori_loop(0, NPAD // 16, zero, 0)

    ones = jnp.ones((16,), jnp.float32)

    def scat(g, c):
        vals = idx_v[pl.ds(g * 16, 16)]
        plsc.addupdate_scatter(hist_v, [vals], ones)
        return c

    lax.fori_loop(0, EPTP // 16, scat, 0)

    pltpu.sync_copy(hist_v, shared_h.at[sid])
    plsc.subcore_barrier()

    off = sid * SLICE
    for p in range(NS):
        pltpu.sync_copy(shared_h.at[p, pl.ds(off, SLICE)], slice_v.at[p])

    def merge(i, c):
        sl = pl.ds(i * 16, 16)
        acc = slice_v[0, sl]
        for p in range(1, NS):
            acc = acc + slice_v[p, sl]
        out_v[sl] = acc
        return c

    lax.fori_loop(0, SLICE // 16, merge, 0)
    pltpu.sync_copy(out_v, deg_hbm.at[cid, pl.ds(off, SLICE)])


_GDN = lax.GatherDimensionNumbers(
    offset_dims=(), collapsed_slice_dims=(0,), start_index_map=(0,))


def _lane_bcast(vec, j):
    idx = jnp.full((16, 1), j, jnp.int32)
    return lax.gather(vec, idx, _GDN, (1,),
                      mode=lax.GatherScatterMode.PROMISE_IN_BOUNDS)


@functools.partial(
    pl.kernel,
    out_type=(jax.ShapeDtypeStruct((NW, 8, 16), jnp.float32),
              jax.ShapeDtypeStruct((NW, 16), jnp.float32)),
    mesh=_MESH,
    compiler_params=pltpu.CompilerParams(needs_layout_passes=False),
    scratch_types=[
        pltpu.VMEM((EPT,), jnp.int32),
        pltpu.VMEM((EPT,), jnp.int32),
        pltpu.VMEM((NC, NPAD), jnp.float32),
        pltpu.VMEM((NPAD,), jnp.float32),
        pltpu.VMEM((K, D), jnp.float32),
        pltpu.VMEM((K, D), jnp.float32),
        pltpu.VMEM((K, D), jnp.float32),
        pltpu.VMEM((K, D), jnp.float32),
        pltpu.VMEM((K, D), jnp.float32),
        pltpu.VMEM((K, D), jnp.float32),
        pltpu.VMEM((8, 16), jnp.float32),
        pltpu.VMEM((16,), jnp.float32),
        pltpu.SemaphoreType.DMA,
        pltpu.SemaphoreType.DMA,
    ],
)
def _edge_kernel(src_hbm, dst_hbm, deg_hbm, a_hbm, b_hbm, e_hbm,
                 acc_hbm, sumw_hbm,
                 src_v, dst_v, deg_v, w_v, a0_buf, b0_buf, e0_buf,
                 a1_buf, b1_buf, e1_buf, acc_v, sumw_v, sem0, sem1):
    cid = lax.axis_index("c")
    sid = lax.axis_index("s")
    wid = cid * NS + sid
    base = wid * EPT

    pltpu.sync_copy(src_hbm.at[pl.ds(base, EPT)], src_v)
    pltpu.sync_copy(dst_hbm.at[pl.ds(base, EPT)], dst_v)
    pltpu.sync_copy(deg_hbm, deg_v)

    def wbody(i, carry):
        d = deg_v[0, pl.ds(i * 16, 16)] + deg_v[1, pl.ds(i * 16, 16)]
        w_v[pl.ds(i * 16, 16)] = jnp.where(
            d > 0.0, 1.0 / jnp.maximum(d, 1.0), 0.0)
        return carry

    lax.fori_loop(0, N // 16, wbody, 0)

    zero16 = jnp.zeros((16,), jnp.float32)
    iota16 = lax.iota(jnp.int32, 16)

    slot0 = (a0_buf, b0_buf, e0_buf, sem0)
    slot1 = (a1_buf, b1_buf, e1_buf, sem1)

    def start(c, slot):
        ab, bb, eb, sem = slot
        pltpu.async_copy(a_hbm.at[src_v.at[pl.ds(c * K, K)]], ab, sem)
        pltpu.async_copy(b_hbm.at[dst_v.at[pl.ds(c * K, K)]], bb, sem)
        pltpu.async_copy(e_hbm.at[pl.ds(base + c * K, K)], eb, sem)

    def drain(slot):
        ab, bb, eb, sem = slot
        pltpu.make_async_copy(a_hbm.at[pl.ds(0, K)], ab, sem).wait()
        pltpu.make_async_copy(a_hbm.at[pl.ds(0, K)], bb, sem).wait()
        pltpu.make_async_copy(a_hbm.at[pl.ds(0, K)], eb, sem).wait()

    def compute(c, slot, carry):
        ab, bb, eb, _ = slot

        def group(g, gcarry):
            accs, comps, sumw = gcarry
            row0 = g * 16
            dvals = plsc.load_gather(dst_v, [iota16 + (c * K + row0)])
            w16 = plsc.load_gather(w_v, [dvals])
            part = [zero16] * 8
            for j in range(16):
                wj = _lane_bcast(w16, j)
                row = row0 + j
                for r in range(8):
                    sl = pl.ds(r * 16, 16)
                    h = jnp.maximum(
                        ab[row, sl] + bb[row, sl] + eb[row, sl], 0.0)
                    u = plsc.bitcast(h, jnp.int32)
                    h = plsc.bitcast((u + 32768) & jnp.int32(-65536),
                                     jnp.float32)
                    part[r] = part[r] + h * wj
            new_a, new_c = [], []
            for r in range(8):
                y = part[r] - comps[r]
                t = accs[r] + y
                new_c.append((t - accs[r]) - y)
                new_a.append(t)
            return tuple(new_a), tuple(new_c), sumw + w16

        return lax.fori_loop(0, K // 16, group, carry)

    start(0, slot0)

    def pair(i, carry):
        c0 = 2 * i
        start(c0 + 1, slot1)
        drain(slot0)
        carry = compute(c0, slot0, carry)
        start(c0 + 2, slot0)
        drain(slot1)
        return compute(c0 + 1, slot1, carry)

    carry = lax.fori_loop(
        0, (NCHUNK - 1) // 2, pair,
        (tuple(zero16 for _ in range(8)), tuple(zero16 for _ in range(8)),
         zero16))
    drain(slot0)
    accs, comps, sumw = compute(NCHUNK - 1, slot0, carry)

    for r in range(8):
        acc_v[r] = accs[r] - comps[r]
    sumw_v[...] = sumw
    pltpu.sync_copy(acc_v, acc_hbm.at[wid])
    pltpu.sync_copy(sumw_v, sumw_hbm.at[wid])


def kernel(x, edge_attr, edge_index, W1, b1, W2, b2, Wr1, br1, Wr2, br2):
    f32 = jnp.float32
    src = edge_index[0]
    dst = edge_index[1]
    W1a = W1[:D]
    W1b = W1[D:2 * D]
    W1c = W1[2 * D:]

    a_mat, b_mat = pl.pallas_call(
        _proj_body,
        grid=(10,),
        in_specs=[
            pl.BlockSpec((N // 10, D), lambda i: (i, 0)),
            pl.BlockSpec((D, D), lambda i: (0, 0)),
            pl.BlockSpec((D, D), lambda i: (0, 0)),
        ],
        out_specs=[
            pl.BlockSpec((N // 10, D), lambda i: (i, 0)),
            pl.BlockSpec((N // 10, D), lambda i: (i, 0)),
        ],
        out_shape=[jax.ShapeDtypeStruct((N, D), f32)] * 2,
    )(x, W1a, W1b)

    eatt = pl.pallas_call(
        _eatt_body,
        grid=(100,),
        in_specs=[
            pl.BlockSpec((E // 100, DE), lambda i: (i, 0)),
            pl.BlockSpec((DE, H), lambda i: (0, 0)),
            pl.BlockSpec((1, H), lambda i: (0, 0)),
        ],
        out_specs=pl.BlockSpec((E // 100, H), lambda i: (i, 0)),
        out_shape=jax.ShapeDtypeStruct((E, H), f32),
    )(edge_attr, W1c, b1.reshape(1, H))

    dst_pad = jnp.concatenate(
        [dst.reshape(NW, EPT),
         jnp.full((NW, EPTP - EPT), N, jnp.int32)], axis=1)
    degp = _deg_kernel(dst_pad)

    acc, sumw = _edge_kernel(src, dst, degp, a_mat, b_mat, eatt)

    out = pl.pallas_call(
        _final_body,
        in_specs=[
            pl.BlockSpec((NW, H), lambda: (0, 0)),
            pl.BlockSpec((NW, 16), lambda: (0, 0)),
            pl.BlockSpec((H, H), lambda: (0, 0)),
            pl.BlockSpec((1, H), lambda: (0, 0)),
            pl.BlockSpec((H, H), lambda: (0, 0)),
            pl.BlockSpec((1, H), lambda: (0, 0)),
            pl.BlockSpec((H, 1), lambda: (0, 0)),
            pl.BlockSpec((1, 1), lambda: (0, 0)),
        ],
        out_specs=pl.BlockSpec((1, 1), lambda: (0, 0)),
        out_shape=jax.ShapeDtypeStruct((1, 1), f32),
    )(acc.reshape(NW, H), sumw, W2, b2.reshape(1, H), Wr1,
      br1.reshape(1, H), Wr2, br2.reshape(1, 1))

    return out[0, 0]


# speedup vs baseline: 1.3664x; 1.0116x over previous
import functools

import jax
import jax.numpy as jnp
from jax import lax
from jax.experimental import pallas as pl
from jax.experimental.pallas import tpu as pltpu
from jax.experimental.pallas import tpu_sc as plsc

N = 10000
E = 320000
D = 128
H = 128
DE = 16

NC = 2
NS = 16
NW = NC * NS
EPT = E // NW
K = 80
NCHUNK = EPT // K
DPC = 128
NDC = (EPT + DPC - 1) // DPC
NPAD = 10240


def _front_body(x_ref, wa_ref, wb_ref, ea_ref, wc_ref, b1_ref,
                a_ref, b_ref, o_ref):
    o_ref[...] = (
        jnp.dot(ea_ref[...], wc_ref[...], preferred_element_type=jnp.float32)
        + b1_ref[...]
    )

    @pl.when(pl.program_id(0) < 10)
    def _():
        xb = x_ref[...]
        a_ref[...] = jnp.dot(xb, wa_ref[...],
                             preferred_element_type=jnp.float32)
        b_ref[...] = jnp.dot(xb, wb_ref[...],
                             preferred_element_type=jnp.float32)


def _bf16r(v):
    return v.astype(jnp.bfloat16).astype(jnp.float32)


def _final_body(acc_ref, sw_ref, w2_ref, b2_ref, wr1_ref, br1_ref, wr2_ref,
                br2_ref, o_ref):
    hi = lax.Precision.HIGHEST
    acc = jnp.sum(acc_ref[...], axis=0, keepdims=True)
    sw = jnp.sum(sw_ref[...])
    hg = (jnp.dot(acc, w2_ref[...], preferred_element_type=jnp.float32,
                  precision=hi) + sw * b2_ref[...]) * (1.0 / N)
    h = jnp.maximum(
        jnp.dot(_bf16r(hg), wr1_ref[...], preferred_element_type=jnp.float32,
                precision=hi) + br1_ref[...], 0.0)
    o_ref[...] = (jnp.dot(_bf16r(h), wr2_ref[...],
                          preferred_element_type=jnp.float32, precision=hi)
                  + br2_ref[...])


_MESH = plsc.VectorSubcoreMesh(core_axis_name="c", subcore_axis_name="s",
                               num_cores=NC, num_subcores=NS)


EPTP = NDC * DPC
SLICE = NPAD // NS


@functools.partial(
    pl.kernel,
    out_type=jax.ShapeDtypeStruct((NC, NPAD), jnp.float32),
    mesh=_MESH,
    compiler_params=pltpu.CompilerParams(needs_layout_passes=False),
    scratch_types=[
        pltpu.VMEM((EPTP,), jnp.int32),
        pltpu.VMEM((NPAD,), jnp.float32),
        pltpu.VMEM_SHARED((NS, NPAD), jnp.float32),
        pltpu.VMEM((NS, SLICE), jnp.float32),
        pltpu.VMEM((SLICE,), jnp.float32),
    ],
)
def _deg_kernel(dstp_hbm, deg_hbm, idx_v, hist_v, shared_h, slice_v, out_v):
    cid = lax.axis_index("c")
    sid = lax.axis_index("s")
    wid = cid * NS + sid

    pltpu.sync_copy(dstp_hbm.at[wid], idx_v)

    def zero(i, c):
        hist_v[pl.ds(i * 16, 16)] = jnp.zeros((16,), jnp.float32)
        return c

    lax.fori_loop(0, NPAD // 16, zero, 0)

    ones = jnp.ones((16,), jnp.float32)

    def scat(g, c):
        vals = idx_v[pl.ds(g * 16, 16)]
        plsc.addupdate_scatter(hist_v, [vals], ones)
        return c

    lax.fori_loop(0, EPTP // 16, scat, 0)

    pltpu.sync_copy(hist_v, shared_h.at[sid])
    plsc.subcore_barrier()

    off = sid * SLICE
    for p in range(NS):
        pltpu.sync_copy(shared_h.at[p, pl.ds(off, SLICE)], slice_v.at[p])

    def merge(i, c):
        sl = pl.ds(i * 16, 16)
        acc = slice_v[0, sl]
        for p in range(1, NS):
            acc = acc + slice_v[p, sl]
        out_v[sl] = acc
        return c

    lax.fori_loop(0, SLICE // 16, merge, 0)
    pltpu.sync_copy(out_v, deg_hbm.at[cid, pl.ds(off, SLICE)])


_GDN = lax.GatherDimensionNumbers(
    offset_dims=(), collapsed_slice_dims=(0,), start_index_map=(0,))


def _lane_bcast(vec, j):
    idx = jnp.full((16, 1), j, jnp.int32)
    return lax.gather(vec, idx, _GDN, (1,),
                      mode=lax.GatherScatterMode.PROMISE_IN_BOUNDS)


@functools.partial(
    pl.kernel,
    out_type=(jax.ShapeDtypeStruct((NW, 8, 16), jnp.float32),
              jax.ShapeDtypeStruct((NW, 16), jnp.float32)),
    mesh=_MESH,
    compiler_params=pltpu.CompilerParams(needs_layout_passes=False),
    scratch_types=[
        pltpu.VMEM((EPT,), jnp.int32),
        pltpu.VMEM((EPT,), jnp.int32),
        pltpu.VMEM((NC, NPAD), jnp.float32),
        pltpu.VMEM((NPAD,), jnp.float32),
        pltpu.VMEM((K, D), jnp.float32),
        pltpu.VMEM((K, D), jnp.float32),
        pltpu.VMEM((K, D), jnp.float32),
        pltpu.VMEM((K, D), jnp.float32),
        pltpu.VMEM((K, D), jnp.float32),
        pltpu.VMEM((K, D), jnp.float32),
        pltpu.VMEM((8, 16), jnp.float32),
        pltpu.VMEM((16,), jnp.float32),
        pltpu.SemaphoreType.DMA,
        pltpu.SemaphoreType.DMA,
    ],
)
def _edge_kernel(src_hbm, dst_hbm, deg_hbm, a_hbm, b_hbm, e_hbm,
                 acc_hbm, sumw_hbm,
                 src_v, dst_v, deg_v, w_v, a0_buf, b0_buf, e0_buf,
                 a1_buf, b1_buf, e1_buf, acc_v, sumw_v, sem0, sem1):
    cid = lax.axis_index("c")
    sid = lax.axis_index("s")
    wid = cid * NS + sid
    base = wid * EPT

    pltpu.sync_copy(src_hbm.at[pl.ds(base, EPT)], src_v)
    pltpu.sync_copy(dst_hbm.at[pl.ds(base, EPT)], dst_v)
    pltpu.sync_copy(deg_hbm, deg_v)

    def wbody(i, carry):
        d = deg_v[0, pl.ds(i * 16, 16)] + deg_v[1, pl.ds(i * 16, 16)]
        w_v[pl.ds(i * 16, 16)] = jnp.where(
            d > 0.0, 1.0 / jnp.maximum(d, 1.0), 0.0)
        return carry

    lax.fori_loop(0, N // 16, wbody, 0)

    zero16 = jnp.zeros((16,), jnp.float32)
    iota16 = lax.iota(jnp.int32, 16)

    slot0 = (a0_buf, b0_buf, e0_buf, sem0)
    slot1 = (a1_buf, b1_buf, e1_buf, sem1)

    def start(c, slot):
        ab, bb, eb, sem = slot
        pltpu.async_copy(a_hbm.at[src_v.at[pl.ds(c * K, K)]], ab, sem)
        pltpu.async_copy(b_hbm.at[dst_v.at[pl.ds(c * K, K)]], bb, sem)
        pltpu.async_copy(e_hbm.at[pl.ds(base + c * K, K)], eb, sem)

    def drain(slot):
        ab, bb, eb, sem = slot
        pltpu.make_async_copy(a_hbm.at[pl.ds(0, K)], ab, sem).wait()
        pltpu.make_async_copy(a_hbm.at[pl.ds(0, K)], bb, sem).wait()
        pltpu.make_async_copy(a_hbm.at[pl.ds(0, K)], eb, sem).wait()

    def compute(c, slot, carry):
        ab, bb, eb, _ = slot

        def group(g, gcarry):
            accs, comps, sumw = gcarry
            row0 = g * 16
            dvals = plsc.load_gather(dst_v, [iota16 + (c * K + row0)])
            w16 = plsc.load_gather(w_v, [dvals])
            part = [zero16] * 8
            for j in range(16):
                wj = _lane_bcast(w16, j)
                row = row0 + j
                for r in range(8):
                    sl = pl.ds(r * 16, 16)
                    h = jnp.maximum(
                        ab[row, sl] + bb[row, sl] + eb[row, sl], 0.0)
                    u = plsc.bitcast(h, jnp.int32)
                    h = plsc.bitcast((u + 32768) & jnp.int32(-65536),
                                     jnp.float32)
                    part[r] = part[r] + h * wj
            new_a, new_c = [], []
            for r in range(8):
                y = part[r] - comps[r]
                t = accs[r] + y
                new_c.append((t - accs[r]) - y)
                new_a.append(t)
            return tuple(new_a), tuple(new_c), sumw + w16

        return lax.fori_loop(0, K // 16, group, carry)

    start(0, slot0)

    def pair(i, carry):
        c0 = 2 * i
        start(c0 + 1, slot1)
        drain(slot0)
        carry = compute(c0, slot0, carry)
        start(c0 + 2, slot0)
        drain(slot1)
        return compute(c0 + 1, slot1, carry)

    carry = lax.fori_loop(
        0, (NCHUNK - 1) // 2, pair,
        (tuple(zero16 for _ in range(8)), tuple(zero16 for _ in range(8)),
         zero16))
    drain(slot0)
    accs, comps, sumw = compute(NCHUNK - 1, slot0, carry)

    for r in range(8):
        acc_v[r] = accs[r] - comps[r]
    sumw_v[...] = sumw
    pltpu.sync_copy(acc_v, acc_hbm.at[wid])
    pltpu.sync_copy(sumw_v, sumw_hbm.at[wid])


def kernel(x, edge_attr, edge_index, W1, b1, W2, b2, Wr1, br1, Wr2, br2):
    f32 = jnp.float32
    src = edge_index[0]
    dst = edge_index[1]
    W1a = W1[:D]
    W1b = W1[D:2 * D]
    W1c = W1[2 * D:]

    clamp10 = lambda i: (jnp.minimum(i, 9), 0)
    a_mat, b_mat, eatt = pl.pallas_call(
        _front_body,
        grid=(100,),
        in_specs=[
            pl.BlockSpec((N // 10, D), clamp10),
            pl.BlockSpec((D, D), lambda i: (0, 0)),
            pl.BlockSpec((D, D), lambda i: (0, 0)),
            pl.BlockSpec((E // 100, DE), lambda i: (i, 0)),
            pl.BlockSpec((DE, H), lambda i: (0, 0)),
            pl.BlockSpec((1, H), lambda i: (0, 0)),
        ],
        out_specs=[
            pl.BlockSpec((N // 10, D), clamp10),
            pl.BlockSpec((N // 10, D), clamp10),
            pl.BlockSpec((E // 100, H), lambda i: (i, 0)),
        ],
        out_shape=[
            jax.ShapeDtypeStruct((N, D), f32),
            jax.ShapeDtypeStruct((N, D), f32),
            jax.ShapeDtypeStruct((E, H), f32),
        ],
    )(x, W1a, W1b, edge_attr, W1c, b1.reshape(1, H))

    dst_pad = jnp.concatenate(
        [dst.reshape(NW, EPT),
         jnp.full((NW, EPTP - EPT), N, jnp.int32)], axis=1)
    degp = _deg_kernel(dst_pad)

    acc, sumw = _edge_kernel(src, dst, degp, a_mat, b_mat, eatt)

    out = pl.pallas_call(
        _final_body,
        in_specs=[
            pl.BlockSpec((NW, H), lambda: (0, 0)),
            pl.BlockSpec((NW, 16), lambda: (0, 0)),
            pl.BlockSpec((H, H), lambda: (0, 0)),
            pl.BlockSpec((1, H), lambda: (0, 0)),
            pl.BlockSpec((H, H), lambda: (0, 0)),
            pl.BlockSpec((1, H), lambda: (0, 0)),
            pl.BlockSpec((H, 1), lambda: (0, 0)),
            pl.BlockSpec((1, 1), lambda: (0, 0)),
        ],
        out_specs=pl.BlockSpec((1, 1), lambda: (0, 0)),
        out_shape=jax.ShapeDtypeStruct((1, 1), f32),
    )(acc.reshape(NW, H), sumw, W2, b2.reshape(1, H), Wr1,
      br1.reshape(1, H), Wr2, br2.reshape(1, 1))

    return out[0, 0]
